# baseline (speedup 1.0000x reference)
import functools

import jax
import jax.numpy as jnp
from jax import lax
from jax.experimental import pallas as pl
from jax.experimental.pallas import tpu as pltpu
from jax.experimental.pallas import tpu_sc as plsc

N_USER = 25000
N_ITEM = 25000
N_ALL = N_USER + N_ITEM
D = 64
HD = 32

L = 16
NC = 2
NS = 16
G = 256
G_R = 128


def _round_up(x, m):
    return (x + m - 1) // m * m


def _zero_spmem(ybuf, rows, s, wc, gsz):
    nfull = wc // gsz
    rem = wc - nfull * gsz

    def zcopy(i, _):
        pltpu.sync_copy(rows, ybuf.at[pl.ds(s * wc + i * gsz, gsz)])
        return 0

    lax.fori_loop(0, nfull, zcopy, 0)
    if rem:
        pltpu.sync_copy(rows.at[pl.ds(0, rem)],
                        ybuf.at[pl.ds(s * wc + nfull * gsz, rem)])


def _zero_rows_vmem(rows, gsz):
    z = jnp.zeros((L,), jnp.float32)

    def zb(i, _):
        for k in range(HD // L):
            rows[i, pl.ds(k * L, L)] = z
        return 0

    lax.fori_loop(0, gsz, zb, 0)


_GATHER_DNUMS = lax.GatherDimensionNumbers(
    offset_dims=(), collapsed_slice_dims=(0,), start_index_map=(0,))


def _lane_broadcast(v16, e):
    return lax.gather(v16, jnp.full((L, 1), e, jnp.int32), _GATHER_DNUMS,
                      slice_sizes=(1,),
                      mode=lax.GatherScatterMode.PROMISE_IN_BOUNDS)


def _transform_src(src3, p, c, sg, gsz):
    for g in range(sg):
        for j in range(gsz // L):
            sl = pl.ds(j * L, L)
            src3[p, g, sl] = src3[p, g, sl] * 2 + c


def _scale_rows(rows, val3, p, g, gsz):
    for j in range(gsz // L):
        v16 = val3[p, g, pl.ds(j * L, L)]
        for e in range(L):
            ee = j * L + e
            vb = _lane_broadcast(v16, e)
            for k in range(HD // L):
                rows[ee, pl.ds(k * L, L)] = rows[ee, pl.ds(k * L, L)] * vb


class _Stream:
    def __init__(self, x_hbm, ybuf, rows_a, rows_b, sem_ga, sem_gb, sem_sa,
                 sem_sb):
        self.x = x_hbm
        self.y = ybuf
        self.rows = (rows_a, rows_b)
        self.sem_g = (sem_ga, sem_gb)
        self.sem_s = (sem_sa, sem_sb)

    def issue_gather(self, src3, p, g, b):
        pltpu.async_copy(self.x.at[src3.at[p, g]], self.rows[b],
                         self.sem_g[b])

    def wait_gather(self, src3, p, g, b):
        pltpu.make_async_copy(self.x.at[src3.at[p, g]], self.rows[b],
                              self.sem_g[b]).wait()

    def issue_scatter(self, dst3, p, g, b):
        pltpu.async_copy(self.rows[b], self.y.at[dst3.at[p, g]],
                         self.sem_s[b], add=True)

    def wait_scatter(self, dst3, p, g, b):
        pltpu.make_async_copy(self.rows[b], self.y.at[dst3.at[p, g]],
                              self.sem_s[b]).wait()


def _edge_phase(streams, src2_h, dst2_h, val2_h, src3, dst3, val3, sem_i,
                s, c, ngrp, sg, gsz):
    nsg = ngrp // sg

    def idx_copies(blk, q):
        row0 = s * ngrp + blk * sg
        return (
            pltpu.make_async_copy(src2_h.at[pl.ds(row0, sg)], src3.at[q],
                                  sem_i),
            pltpu.make_async_copy(dst2_h.at[pl.ds(row0, sg)], dst3.at[q],
                                  sem_i),
            pltpu.make_async_copy(val2_h.at[pl.ds(row0, sg)], val3.at[q],
                                  sem_i),
        )

    for d in idx_copies(0, 0):
        d.start()

    def outer(sb, _):
        p = lax.rem(sb, 2)
        for d in idx_copies(sb, p):
            d.wait()

        @pl.when(sb + 1 < nsg)
        def _():
            for d in idx_copies(sb + 1, 1 - p):
                d.start()

        _transform_src(src3, p, c, sg, gsz)
        for st in streams:
            st.issue_gather(src3, p, 0, 0)

        def inner(t, _):
            g0 = 2 * t
            g1 = 2 * t + 1

            @pl.when(t > 0)
            def _():
                for st in streams:
                    st.wait_scatter(dst3, p, g0 - 1, 1)
            for st in streams:
                st.issue_gather(src3, p, g1, 1)
            for st in streams:
                st.wait_gather(src3, p, g0, 0)
            for st in streams:
                _scale_rows(st.rows[0], val3, p, g0, gsz)
                st.issue_scatter(dst3, p, g0, 0)

            @pl.when(t < sg // 2 - 1)
            def _():
                for st in streams:
                    st.wait_scatter(dst3, p, g0, 0)
                    st.issue_gather(src3, p, g0 + 2, 0)
            for st in streams:
                st.wait_gather(src3, p, g1, 1)
            for st in streams:
                _scale_rows(st.rows[1], val3, p, g1, gsz)
                st.issue_scatter(dst3, p, g1, 1)
            return 0

        lax.fori_loop(0, sg // 2, inner, 0)
        for st in streams:
            st.wait_scatter(dst3, p, sg - 2, 0)
            st.wait_scatter(dst3, p, sg - 1, 1)
        return 0

    lax.fori_loop(0, nsg, outer, 0)


def _writeback(ybuf, out_hbm, s, c, wc):
    pltpu.sync_copy(ybuf.at[pl.ds(s * wc, wc)],
                    out_hbm.at[pl.ds(s * wc, wc), c])


def _npad(n):
    return _round_up(n, 128)


def _make_spmm1(ndst, epad, sg, gsz):
    npad = _npad(ndst)
    wc = npad // NS
    ngrp = epad // NS // gsz
    assert npad % NS == 0 and ngrp % sg == 0 and sg % 2 == 0
    mesh = plsc.VectorSubcoreMesh(core_axis_name="c", subcore_axis_name="s")

    @functools.partial(
        pl.kernel, mesh=mesh,
        compiler_params=pltpu.CompilerParams(use_tc_tiling_on_sc=False),
        out_type=jax.ShapeDtypeStruct((npad, NC, HD), jnp.float32),
        scratch_types=[
            pltpu.VMEM_SHARED((npad, HD), jnp.float32),
            pltpu.VMEM((2, sg, gsz), jnp.int32),
            pltpu.VMEM((2, sg, gsz), jnp.int32),
            pltpu.VMEM((2, sg, gsz), jnp.float32),
            pltpu.VMEM((gsz, HD), jnp.float32),
            pltpu.VMEM((gsz, HD), jnp.float32),
            pltpu.SemaphoreType.DMA,
            pltpu.SemaphoreType.DMA,
            pltpu.SemaphoreType.DMA,
            pltpu.SemaphoreType.DMA,
            pltpu.SemaphoreType.DMA,
        ])
    def k(x_hbm, src_h, dst_h, val_h, out_hbm, ybuf, src3, dst3, val3,
          rows_a, rows_b, sem_i, sem_ga, sem_gb, sem_sa, sem_sb):
        c = lax.axis_index("c")
        s = lax.axis_index("s")
        _zero_rows_vmem(rows_a, gsz)
        _zero_spmem(ybuf, rows_a, s, wc, gsz)
        plsc.subcore_barrier()
        st = _Stream(x_hbm, ybuf, rows_a, rows_b, sem_ga, sem_gb, sem_sa,
                     sem_sb)
        _edge_phase([st], src_h, dst_h, val_h, src3, dst3, val3, sem_i, s,
                    c, ngrp, sg, gsz)
        plsc.subcore_barrier()
        _writeback(ybuf, out_hbm, s, c, wc)

    return k


def _make_spmm2(ndst, epad, sg, gsz, shared_edges):
    npad = _npad(ndst)
    wc = npad // NS
    ngrp = epad // NS // gsz
    assert npad % NS == 0 and ngrp % sg == 0 and sg % 2 == 0
    mesh = plsc.VectorSubcoreMesh(core_axis_name="c", subcore_axis_name="s")

    nrows = 4 if shared_edges else 2
    scratch = [
        pltpu.VMEM_SHARED((npad, HD), jnp.float32),
        pltpu.VMEM_SHARED((npad, HD), jnp.float32),
        pltpu.VMEM((2, sg, gsz), jnp.int32),
        pltpu.VMEM((2, sg, gsz), jnp.int32),
        pltpu.VMEM((2, sg, gsz), jnp.float32),
    ] + [pltpu.VMEM((gsz, HD), jnp.float32)] * nrows \
      + [pltpu.SemaphoreType.DMA] * 9
    out_type = [jax.ShapeDtypeStruct((npad, NC, HD), jnp.float32),
                jax.ShapeDtypeStruct((npad, NC, HD), jnp.float32)]

    if shared_edges:
        @functools.partial(
            pl.kernel, mesh=mesh, out_type=out_type, scratch_types=scratch,
            compiler_params=pltpu.CompilerParams(use_tc_tiling_on_sc=False))
        def k(xa_hbm, xb_hbm, src_h, dst_h, val_h, outa, outb, ybufa, ybufb,
              src3, dst3, val3, rows_aa, rows_ab, rows_ba, rows_bb, sem_i,
              sem_ga1, sem_gb1, sem_sa1, sem_sb1, sem_ga2, sem_gb2, sem_sa2,
              sem_sb2):
            c = lax.axis_index("c")
            s = lax.axis_index("s")
            _zero_rows_vmem(rows_aa, gsz)
            _zero_spmem(ybufa, rows_aa, s, wc, gsz)
            _zero_spmem(ybufb, rows_aa, s, wc, gsz)
            plsc.subcore_barrier()
            sta = _Stream(xa_hbm, ybufa, rows_aa, rows_ab, sem_ga1, sem_gb1,
                          sem_sa1, sem_sb1)
            stb = _Stream(xb_hbm, ybufb, rows_ba, rows_bb, sem_ga2, sem_gb2,
                          sem_sa2, sem_sb2)
            _edge_phase([sta, stb], src_h, dst_h, val_h, src3, dst3, val3,
                        sem_i, s, c, ngrp, sg, gsz)
            plsc.subcore_barrier()
            _writeback(ybufa, outa, s, c, wc)
            _writeback(ybufb, outb, s, c, wc)
    else:
        @functools.partial(
            pl.kernel, mesh=mesh, out_type=out_type, scratch_types=scratch,
            compiler_params=pltpu.CompilerParams(use_tc_tiling_on_sc=False))
        def k(xa_hbm, srca_h, dsta_h, vala_h, xb_hbm, srcb_h, dstb_h,
              valb_h, outa, outb, ybufa, ybufb, src3, dst3, val3, rows_aa,
              rows_ab, sem_i, sem_ga1, sem_gb1, sem_sa1,
              sem_sb1, sem_ga2, sem_gb2, sem_sa2, sem_sb2):
            c = lax.axis_index("c")
            s = lax.axis_index("s")
            _zero_rows_vmem(rows_aa, gsz)
            _zero_spmem(ybufa, rows_aa, s, wc, gsz)
            _zero_spmem(ybufb, rows_aa, s, wc, gsz)
            plsc.subcore_barrier()
            sta = _Stream(xa_hbm, ybufa, rows_aa, rows_ab, sem_ga1, sem_gb1,
                          sem_sa1, sem_sb1)
            stb = _Stream(xb_hbm, ybufb, rows_aa, rows_ab, sem_ga2, sem_gb2,
                          sem_sa2, sem_sb2)
            _edge_phase([sta], srca_h, dsta_h, vala_h, src3, dst3, val3,
                        sem_i, s, c, ngrp, sg, gsz)
            _edge_phase([stb], srcb_h, dstb_h, valb_h, src3, dst3, val3,
                        sem_i, s, c, ngrp, sg, gsz)
            plsc.subcore_barrier()
            _writeback(ybufa, outa, s, c, wc)
            _writeback(ybufb, outb, s, c, wc)

    return k


_BA = 1000
_BC = 2000


def _stage_a_body(vf, tf, ie, Wv, bv, Wt, bt, Wgv, bgv, Wgt, bgt,
                  img_o, txt_o):
    imf = jnp.dot(vf[...], Wv[...], preferred_element_type=jnp.float32)
    imf = imf + bv[...]
    txf = jnp.dot(tf[...], Wt[...], preferred_element_type=jnp.float32)
    txf = txf + bt[...]
    gi = jax.nn.sigmoid(
        jnp.dot(imf, Wgv[...], preferred_element_type=jnp.float32) + bgv[...])
    gt = jax.nn.sigmoid(
        jnp.dot(txf, Wgt[...], preferred_element_type=jnp.float32) + bgt[...])
    img_o[...] = ie[...] * gi
    txt_o[...] = ie[...] * gt


def _stage_a(v_feat, t_feat, item_emb, Wv, bv, Wt, bt, Wgv, bgv, Wgt, bgt):
    grid = (N_ITEM // _BA,)
    row = lambda i: (i, 0)
    full = lambda i: (0, 0)
    return pl.pallas_call(
        _stage_a_body,
        grid=grid,
        in_specs=[
            pl.BlockSpec((_BA, 4096), row),
            pl.BlockSpec((_BA, 384), row),
            pl.BlockSpec((_BA, D), row),
            pl.BlockSpec((4096, D), full),
            pl.BlockSpec((1, D), full),
            pl.BlockSpec((384, D), full),
            pl.BlockSpec((1, D), full),
            pl.BlockSpec((D, D), full),
            pl.BlockSpec((1, D), full),
            pl.BlockSpec((D, D), full),
            pl.BlockSpec((1, D), full),
        ],
        out_specs=[
            pl.BlockSpec((_BA, D), row),
            pl.BlockSpec((_BA, D), row),
        ],
        out_shape=[
            jax.ShapeDtypeStruct((N_ITEM, D), jnp.float32),
            jax.ShapeDtypeStruct((N_ITEM, D), jnp.float32),
        ],
    )(v_feat, t_feat, item_emb, Wv, bv.reshape(1, D), Wt, bt.reshape(1, D),
      Wgv, bgv.reshape(1, D), Wgt, bgt.reshape(1, D))


def _stage_c_body(ego, h1, h2, ie, te, Wq1, bq1, wq2, Wip, bip, Wtp, btp,
                  out):
    content = (ego[...] + h1[...] + h2[...]) * (1.0 / 3.0)
    iev = ie[...]
    tev = te[...]
    q1 = Wq1[...]
    b1 = bq1[...]
    q2 = wq2[...]
    ai = jnp.sum(jnp.tanh(
        jnp.dot(iev, q1, preferred_element_type=jnp.float32) + b1) * q2,
        axis=-1, keepdims=True)
    at = jnp.sum(jnp.tanh(
        jnp.dot(tev, q1, preferred_element_type=jnp.float32) + b1) * q2,
        axis=-1, keepdims=True)
    wi = jax.nn.sigmoid(ai - at)
    common = wi * iev + (1.0 - wi) * tev
    gi = jax.nn.sigmoid(
        jnp.dot(content, Wip[...], preferred_element_type=jnp.float32)
        + bip[...])
    gt = jax.nn.sigmoid(
        jnp.dot(content, Wtp[...], preferred_element_type=jnp.float32)
        + btp[...])
    sep = (iev - common) * gi + (tev - common) * gt
    out[...] = content + (sep + common) * (1.0 / 3.0)


def _stage_c(ego, h1, h2, ie, te, Wq1, bq1, Wq2, Wip, bip, Wtp, btp):
    grid = (N_ALL // _BC,)
    row = lambda i: (i, 0)
    full = lambda i: (0, 0)
    return pl.pallas_call(
        _stage_c_body,
        grid=grid,
        in_specs=[
            pl.BlockSpec((_BC, D), row),
            pl.BlockSpec((_BC, D), row),
            pl.BlockSpec((_BC, D), row),
            pl.BlockSpec((_BC, D), row),
            pl.BlockSpec((_BC, D), row),
            pl.BlockSpec((D, D), full),
            pl.BlockSpec((1, D), full),
            pl.BlockSpec((1, D), full),
            pl.BlockSpec((D, D), full),
            pl.BlockSpec((1, D), full),
            pl.BlockSpec((D, D), full),
            pl.BlockSpec((1, D), full),
        ],
        out_specs=pl.BlockSpec((_BC, D), row),
        out_shape=jax.ShapeDtypeStruct((N_ALL, D), jnp.float32),
    )(ego, h1, h2, ie, te, Wq1, bq1.reshape(1, D), Wq2.reshape(1, D),
      Wip, bip.reshape(1, D), Wtp, btp.reshape(1, D))


_EPAD_ADJ = 819200
_EPAD_KNN = 262144
_EPAD_R = 409600

_NP_ALL = _npad(N_ALL)
_NP_ITEM = _npad(N_ITEM)
_NP_USER = _npad(N_USER)

_spmm_adj = _make_spmm1(N_ALL, _EPAD_ADJ, 8, G)
_spmm_knn = _make_spmm2(N_ITEM, _EPAD_KNN, 8, G, shared_edges=False)
_spmm_r = _make_spmm2(N_USER, _EPAD_R, 8, G_R, shared_edges=True)


def _pad_edges(idx, val, epad, gsz):
    e = val.shape[0]
    pad = epad - e
    src = jnp.pad(idx[1], (0, pad)).reshape(epad // gsz, gsz)
    dst = jnp.pad(idx[0], (0, pad)).reshape(epad // gsz, gsz)
    v = jnp.pad(val, (0, pad)).reshape(epad // gsz, gsz)
    return src, dst, v


def _as_sc_rows(x, npad):
    n = x.shape[0]
    if n < npad:
        x = jnp.pad(x, ((0, npad - n), (0, 0)))
    return x.reshape(2 * npad, HD)


def _from_sc(y):
    return y.reshape(y.shape[0], D)


def kernel(user_emb, item_emb, v_feat, t_feat, adj_idx, adj_val, R_idx,
           R_val, image_adj_idx, image_adj_val, text_adj_idx, text_adj_val,
           Wv, bv, Wt, bt, Wgv, bgv, Wgt, bgt, Wq1, bq1, Wq2, Wip, bip, Wtp,
           btp):
    ego = jnp.concatenate([user_emb, item_emb], axis=0)
    a_src, a_dst, a_val = _pad_edges(adj_idx, adj_val, _EPAD_ADJ, G)
    h1 = _from_sc(_spmm_adj(_as_sc_rows(ego, _NP_ALL), a_src, a_dst, a_val))
    h2 = _from_sc(_spmm_adj(h1.reshape(2 * _NP_ALL, HD), a_src, a_dst,
                            a_val))

    image_item, text_item = _stage_a(v_feat, t_feat, item_emb, Wv, bv, Wt,
                                     bt, Wgv, bgv, Wgt, bgt)

    i_src, i_dst, i_val = _pad_edges(image_adj_idx, image_adj_val, _EPAD_KNN,
                                     G)
    t_src, t_dst, t_val = _pad_edges(text_adj_idx, text_adj_val, _EPAD_KNN, G)
    (i_src, i_dst, i_val, t_src, t_dst, t_val, h2) = lax.optimization_barrier(
        (i_src, i_dst, i_val, t_src, t_dst, t_val, h2))
    ii, ti = _spmm_knn(_as_sc_rows(image_item, _NP_ITEM), i_src, i_dst,
                       i_val, _as_sc_rows(text_item, _NP_ITEM), t_src,
                       t_dst, t_val)
    ii = _from_sc(ii)
    ti = _from_sc(ti)

    r_src, r_dst, r_val = _pad_edges(R_idx, R_val, _EPAD_R, G_R)
    iu, tu = _spmm_r(ii.reshape(2 * _NP_ITEM, HD),
                     ti.reshape(2 * _NP_ITEM, HD), r_src, r_dst, r_val)
    iu = _from_sc(iu)
    tu = _from_sc(tu)

    ie = jnp.concatenate([iu[:N_USER], ii[:N_ITEM]], axis=0)
    te = jnp.concatenate([tu[:N_USER], ti[:N_ITEM]], axis=0)
    out = _stage_c(ego, h1[:N_ALL], h2[:N_ALL], ie, te, Wq1, bq1, Wq2, Wip,
                   bip, Wtp, btp)
    return out[:N_USER], out[N_USER:]


# speedup vs baseline: 1.0458x; 1.0458x over previous
import functools

import jax
import jax.numpy as jnp
from jax import lax
from jax.experimental import pallas as pl
from jax.experimental.pallas import tpu as pltpu
from jax.experimental.pallas import tpu_sc as plsc

N_USER = 25000
N_ITEM = 25000
N_ALL = N_USER + N_ITEM
D = 64
HD = 32

L = 16
NC = 2
NS = 16
G = 256
G_R = 128


def _round_up(x, m):
    return (x + m - 1) // m * m


def _zero_spmem(ybuf, rows, s, wc, gsz):
    nfull = wc // gsz
    rem = wc - nfull * gsz

    def zcopy(i, _):
        pltpu.sync_copy(rows, ybuf.at[pl.ds(s * wc + i * gsz, gsz)])
        return 0

    lax.fori_loop(0, nfull, zcopy, 0)
    if rem:
        pltpu.sync_copy(rows.at[pl.ds(0, rem)],
                        ybuf.at[pl.ds(s * wc + nfull * gsz, rem)])


def _zero_rows_vmem(rows, gsz):
    z = jnp.zeros((L,), jnp.float32)

    def zb(i, _):
        for k in range(HD // L):
            rows[i, pl.ds(k * L, L)] = z
        return 0

    lax.fori_loop(0, gsz, zb, 0)


_GATHER_DNUMS = lax.GatherDimensionNumbers(
    offset_dims=(), collapsed_slice_dims=(0,), start_index_map=(0,))


def _lane_broadcast(v16, e):
    return lax.gather(v16, jnp.full((L, 1), e, jnp.int32), _GATHER_DNUMS,
                      slice_sizes=(1,),
                      mode=lax.GatherScatterMode.PROMISE_IN_BOUNDS)


def _transform_src(src3, p, c, sg, gsz):
    for g in range(sg):
        for j in range(gsz // L):
            sl = pl.ds(j * L, L)
            src3[p, g, sl] = src3[p, g, sl] * 2 + c


def _scale_rows(rows, val3, p, g, gsz):
    for j in range(gsz // L):
        v16 = val3[p, g, pl.ds(j * L, L)]
        for e in range(L):
            ee = j * L + e
            vb = _lane_broadcast(v16, e)
            for k in range(HD // L):
                rows[ee, pl.ds(k * L, L)] = rows[ee, pl.ds(k * L, L)] * vb


class _Stream:
    def __init__(self, x_hbm, ybuf, rows_a, rows_b, sem_ga, sem_gb, sem_sa,
                 sem_sb):
        self.x = x_hbm
        self.y = ybuf
        self.rows = (rows_a, rows_b)
        self.sem_g = (sem_ga, sem_gb)
        self.sem_s = (sem_sa, sem_sb)

    def issue_gather(self, src3, p, g, b):
        pltpu.async_copy(self.x.at[src3.at[p, g]], self.rows[b],
                         self.sem_g[b])

    def wait_gather(self, src3, p, g, b):
        pltpu.make_async_copy(self.x.at[src3.at[p, g]], self.rows[b],
                              self.sem_g[b]).wait()

    def issue_scatter(self, dst3, p, g, b):
        pltpu.async_copy(self.rows[b], self.y.at[dst3.at[p, g]],
                         self.sem_s[b], add=True)

    def wait_scatter(self, dst3, p, g, b):
        pltpu.make_async_copy(self.rows[b], self.y.at[dst3.at[p, g]],
                              self.sem_s[b]).wait()


def _edge_phase(streams, src2_h, dst2_h, val2_h, src3, dst3, val3, sem_i,
                s, c, ngrp, sg, gsz):
    nsg = ngrp // sg

    def idx_copies(blk, q):
        row0 = s * ngrp + blk * sg
        return (
            pltpu.make_async_copy(src2_h.at[pl.ds(row0, sg)], src3.at[q],
                                  sem_i),
            pltpu.make_async_copy(dst2_h.at[pl.ds(row0, sg)], dst3.at[q],
                                  sem_i),
            pltpu.make_async_copy(val2_h.at[pl.ds(row0, sg)], val3.at[q],
                                  sem_i),
        )

    for d in idx_copies(0, 0):
        d.start()

    def outer(sb, _):
        p = lax.rem(sb, 2)
        for d in idx_copies(sb, p):
            d.wait()

        @pl.when(sb + 1 < nsg)
        def _():
            for d in idx_copies(sb + 1, 1 - p):
                d.start()

        _transform_src(src3, p, c, sg, gsz)
        for st in streams:
            st.issue_gather(src3, p, 0, 0)

        def inner(t, _):
            g0 = 2 * t
            g1 = 2 * t + 1

            @pl.when(t > 0)
            def _():
                for st in streams:
                    st.wait_scatter(dst3, p, g0 - 1, 1)
            for st in streams:
                st.issue_gather(src3, p, g1, 1)
            for st in streams:
                st.wait_gather(src3, p, g0, 0)
            for st in streams:
                _scale_rows(st.rows[0], val3, p, g0, gsz)
                st.issue_scatter(dst3, p, g0, 0)

            @pl.when(t < sg // 2 - 1)
            def _():
                for st in streams:
                    st.wait_scatter(dst3, p, g0, 0)
                    st.issue_gather(src3, p, g0 + 2, 0)
            for st in streams:
                st.wait_gather(src3, p, g1, 1)
            for st in streams:
                _scale_rows(st.rows[1], val3, p, g1, gsz)
                st.issue_scatter(dst3, p, g1, 1)
            return 0

        lax.fori_loop(0, sg // 2, inner, 0)
        for st in streams:
            st.wait_scatter(dst3, p, sg - 2, 0)
            st.wait_scatter(dst3, p, sg - 1, 1)
        return 0

    lax.fori_loop(0, nsg, outer, 0)


def _writeback(ybuf, out_hbm, s, c, wc):
    pltpu.sync_copy(ybuf.at[pl.ds(s * wc, wc)],
                    out_hbm.at[pl.ds(s * wc, wc), c])


def _npad(n):
    return _round_up(n, 128)


def _make_spmm1(ndst, epad, sg, gsz):
    npad = _npad(ndst)
    wc = npad // NS
    ngrp = epad // NS // gsz
    assert npad % NS == 0 and ngrp % sg == 0 and sg % 2 == 0
    mesh = plsc.VectorSubcoreMesh(core_axis_name="c", subcore_axis_name="s")

    @functools.partial(
        pl.kernel, mesh=mesh,
        compiler_params=pltpu.CompilerParams(use_tc_tiling_on_sc=False),
        out_type=jax.ShapeDtypeStruct((npad, NC, HD), jnp.float32),
        scratch_types=[
            pltpu.VMEM_SHARED((npad, HD), jnp.float32),
            pltpu.VMEM((2, sg, gsz), jnp.int32),
            pltpu.VMEM((2, sg, gsz), jnp.int32),
            pltpu.VMEM((2, sg, gsz), jnp.float32),
            pltpu.VMEM((gsz, HD), jnp.float32),
            pltpu.VMEM((gsz, HD), jnp.float32),
            pltpu.SemaphoreType.DMA,
            pltpu.SemaphoreType.DMA,
            pltpu.SemaphoreType.DMA,
            pltpu.SemaphoreType.DMA,
            pltpu.SemaphoreType.DMA,
        ])
    def k(x_hbm, src_h, dst_h, val_h, out_hbm, ybuf, src3, dst3, val3,
          rows_a, rows_b, sem_i, sem_ga, sem_gb, sem_sa, sem_sb):
        c = lax.axis_index("c")
        s = lax.axis_index("s")
        _zero_rows_vmem(rows_a, gsz)
        _zero_spmem(ybuf, rows_a, s, wc, gsz)
        plsc.subcore_barrier()
        st = _Stream(x_hbm, ybuf, rows_a, rows_b, sem_ga, sem_gb, sem_sa,
                     sem_sb)
        _edge_phase([st], src_h, dst_h, val_h, src3, dst3, val3, sem_i, s,
                    c, ngrp, sg, gsz)
        plsc.subcore_barrier()
        _writeback(ybuf, out_hbm, s, c, wc)

    return k


def _make_spmm2(ndst, epad, sg, gsz, shared_edges):
    npad = _npad(ndst)
    wc = npad // NS
    ngrp = epad // NS // gsz
    assert npad % NS == 0 and ngrp % sg == 0 and sg % 2 == 0
    mesh = plsc.VectorSubcoreMesh(core_axis_name="c", subcore_axis_name="s")

    nrows = 4 if shared_edges else 2
    scratch = [
        pltpu.VMEM_SHARED((npad, HD), jnp.float32),
        pltpu.VMEM_SHARED((npad, HD), jnp.float32),
        pltpu.VMEM((2, sg, gsz), jnp.int32),
        pltpu.VMEM((2, sg, gsz), jnp.int32),
        pltpu.VMEM((2, sg, gsz), jnp.float32),
    ] + [pltpu.VMEM((gsz, HD), jnp.float32)] * nrows \
      + [pltpu.SemaphoreType.DMA] * 9
    out_type = [jax.ShapeDtypeStruct((npad, NC, HD), jnp.float32),
                jax.ShapeDtypeStruct((npad, NC, HD), jnp.float32)]

    if shared_edges:
        @functools.partial(
            pl.kernel, mesh=mesh, out_type=out_type, scratch_types=scratch,
            compiler_params=pltpu.CompilerParams(use_tc_tiling_on_sc=False))
        def k(xa_hbm, xb_hbm, src_h, dst_h, val_h, outa, outb, ybufa, ybufb,
              src3, dst3, val3, rows_aa, rows_ab, rows_ba, rows_bb, sem_i,
              sem_ga1, sem_gb1, sem_sa1, sem_sb1, sem_ga2, sem_gb2, sem_sa2,
              sem_sb2):
            c = lax.axis_index("c")
            s = lax.axis_index("s")
            _zero_rows_vmem(rows_aa, gsz)
            _zero_spmem(ybufa, rows_aa, s, wc, gsz)
            _zero_spmem(ybufb, rows_aa, s, wc, gsz)
            plsc.subcore_barrier()
            sta = _Stream(xa_hbm, ybufa, rows_aa, rows_ab, sem_ga1, sem_gb1,
                          sem_sa1, sem_sb1)
            stb = _Stream(xb_hbm, ybufb, rows_ba, rows_bb, sem_ga2, sem_gb2,
                          sem_sa2, sem_sb2)
            _edge_phase([sta, stb], src_h, dst_h, val_h, src3, dst3, val3,
                        sem_i, s, c, ngrp, sg, gsz)
            plsc.subcore_barrier()
            _writeback(ybufa, outa, s, c, wc)
            _writeback(ybufb, outb, s, c, wc)
    else:
        @functools.partial(
            pl.kernel, mesh=mesh, out_type=out_type, scratch_types=scratch,
            compiler_params=pltpu.CompilerParams(use_tc_tiling_on_sc=False))
        def k(xa_hbm, srca_h, dsta_h, vala_h, xb_hbm, srcb_h, dstb_h,
              valb_h, outa, outb, ybufa, ybufb, src3, dst3, val3, rows_aa,
              rows_ab, sem_i, sem_ga1, sem_gb1, sem_sa1,
              sem_sb1, sem_ga2, sem_gb2, sem_sa2, sem_sb2):
            c = lax.axis_index("c")
            s = lax.axis_index("s")
            _zero_rows_vmem(rows_aa, gsz)
            _zero_spmem(ybufa, rows_aa, s, wc, gsz)
            _zero_spmem(ybufb, rows_aa, s, wc, gsz)
            plsc.subcore_barrier()
            sta = _Stream(xa_hbm, ybufa, rows_aa, rows_ab, sem_ga1, sem_gb1,
                          sem_sa1, sem_sb1)
            stb = _Stream(xb_hbm, ybufb, rows_aa, rows_ab, sem_ga2, sem_gb2,
                          sem_sa2, sem_sb2)
            _edge_phase([sta], srca_h, dsta_h, vala_h, src3, dst3, val3,
                        sem_i, s, c, ngrp, sg, gsz)
            _edge_phase([stb], srcb_h, dstb_h, valb_h, src3, dst3, val3,
                        sem_i, s, c, ngrp, sg, gsz)
            plsc.subcore_barrier()
            _writeback(ybufa, outa, s, c, wc)
            _writeback(ybufb, outb, s, c, wc)

    return k


_BA = 1000
_BC = 2000


def _stage_a_body(vf, tf, ie, Wv, bv, Wt, bt, Wgv, bgv, Wgt, bgt,
                  img_o, txt_o):
    imf = jnp.dot(vf[...], Wv[...], preferred_element_type=jnp.float32)
    imf = imf + bv[...]
    txf = jnp.dot(tf[...], Wt[...], preferred_element_type=jnp.float32)
    txf = txf + bt[...]
    gi = jax.nn.sigmoid(
        jnp.dot(imf, Wgv[...], preferred_element_type=jnp.float32) + bgv[...])
    gt = jax.nn.sigmoid(
        jnp.dot(txf, Wgt[...], preferred_element_type=jnp.float32) + bgt[...])
    img_o[...] = ie[...] * gi
    txt_o[...] = ie[...] * gt


def _stage_a(v_feat, t_feat, item_emb, Wv, bv, Wt, bt, Wgv, bgv, Wgt, bgt):
    grid = (N_ITEM // _BA,)
    row = lambda i: (i, 0)
    full = lambda i: (0, 0)
    return pl.pallas_call(
        _stage_a_body,
        grid=grid,
        in_specs=[
            pl.BlockSpec((_BA, 4096), row),
            pl.BlockSpec((_BA, 384), row),
            pl.BlockSpec((_BA, D), row),
            pl.BlockSpec((4096, D), full),
            pl.BlockSpec((1, D), full),
            pl.BlockSpec((384, D), full),
            pl.BlockSpec((1, D), full),
            pl.BlockSpec((D, D), full),
            pl.BlockSpec((1, D), full),
            pl.BlockSpec((D, D), full),
            pl.BlockSpec((1, D), full),
        ],
        out_specs=[
            pl.BlockSpec((_BA, D), row),
            pl.BlockSpec((_BA, D), row),
        ],
        out_shape=[
            jax.ShapeDtypeStruct((N_ITEM, D), jnp.float32),
            jax.ShapeDtypeStruct((N_ITEM, D), jnp.float32),
        ],
    )(v_feat, t_feat, item_emb, Wv, bv.reshape(1, D), Wt, bt.reshape(1, D),
      Wgv, bgv.reshape(1, D), Wgt, bgt.reshape(1, D))


def _stage_c_body(ego, h1, h2, ie, te, Wq1, bq1, wq2, Wip, bip, Wtp, btp,
                  out):
    content = (ego[...] + h1[...] + h2[...]) * (1.0 / 3.0)
    iev = ie[...]
    tev = te[...]
    q1 = Wq1[...]
    b1 = bq1[...]
    q2 = wq2[...]
    ai = jnp.sum(jnp.tanh(
        jnp.dot(iev, q1, preferred_element_type=jnp.float32) + b1) * q2,
        axis=-1, keepdims=True)
    at = jnp.sum(jnp.tanh(
        jnp.dot(tev, q1, preferred_element_type=jnp.float32) + b1) * q2,
        axis=-1, keepdims=True)
    wi = jax.nn.sigmoid(ai - at)
    common = wi * iev + (1.0 - wi) * tev
    gi = jax.nn.sigmoid(
        jnp.dot(content, Wip[...], preferred_element_type=jnp.float32)
        + bip[...])
    gt = jax.nn.sigmoid(
        jnp.dot(content, Wtp[...], preferred_element_type=jnp.float32)
        + btp[...])
    sep = (iev - common) * gi + (tev - common) * gt
    out[...] = content + (sep + common) * (1.0 / 3.0)


def _stage_c(ego, h1, h2, ie, te, Wq1, bq1, Wq2, Wip, bip, Wtp, btp):
    grid = (N_ALL // _BC,)
    row = lambda i: (i, 0)
    full = lambda i: (0, 0)
    return pl.pallas_call(
        _stage_c_body,
        grid=grid,
        in_specs=[
            pl.BlockSpec((_BC, D), row),
            pl.BlockSpec((_BC, D), row),
            pl.BlockSpec((_BC, D), row),
            pl.BlockSpec((_BC, D), row),
            pl.BlockSpec((_BC, D), row),
            pl.BlockSpec((D, D), full),
            pl.BlockSpec((1, D), full),
            pl.BlockSpec((1, D), full),
            pl.BlockSpec((D, D), full),
            pl.BlockSpec((1, D), full),
            pl.BlockSpec((D, D), full),
            pl.BlockSpec((1, D), full),
        ],
        out_specs=pl.BlockSpec((_BC, D), row),
        out_shape=jax.ShapeDtypeStruct((N_ALL, D), jnp.float32),
    )(ego, h1, h2, ie, te, Wq1, bq1.reshape(1, D), Wq2.reshape(1, D),
      Wip, bip.reshape(1, D), Wtp, btp.reshape(1, D))


_EPAD_ADJ = 819200
_EPAD_KNN = 262144
_EPAD_R = 409600

_NP_ALL = _npad(N_ALL)
_NP_ITEM = _npad(N_ITEM)
_NP_USER = _npad(N_USER)

_spmm_adj = _make_spmm1(N_ALL, _EPAD_ADJ, 8, G)
_spmm_knn = _make_spmm2(N_ITEM, _EPAD_KNN, 8, G, shared_edges=False)
_spmm_r = _make_spmm2(N_USER, _EPAD_R, 8, G_R, shared_edges=True)


def _pad_edges(idx, val, epad, gsz):
    e = val.shape[0]
    pad = epad - e
    src = jnp.pad(idx[1], (0, pad)).reshape(epad // gsz, gsz)
    dst = jnp.pad(idx[0], (0, pad)).reshape(epad // gsz, gsz)
    v = jnp.pad(val, (0, pad)).reshape(epad // gsz, gsz)
    return src, dst, v


def _as_sc_rows(x, npad):
    n = x.shape[0]
    if n < npad:
        x = jnp.pad(x, ((0, npad - n), (0, 0)))
    return x.reshape(2 * npad, HD)


def _from_sc(y):
    return y.reshape(y.shape[0], D)


def kernel(user_emb, item_emb, v_feat, t_feat, adj_idx, adj_val, R_idx,
           R_val, image_adj_idx, image_adj_val, text_adj_idx, text_adj_val,
           Wv, bv, Wt, bt, Wgv, bgv, Wgt, bgt, Wq1, bq1, Wq2, Wip, bip, Wtp,
           btp):
    ego = jnp.concatenate([user_emb, item_emb], axis=0)
    a_src, a_dst, a_val = _pad_edges(adj_idx, adj_val, _EPAD_ADJ, G)
    h1 = _from_sc(_spmm_adj(_as_sc_rows(ego, _NP_ALL), a_src, a_dst, a_val))
    h2 = _from_sc(_spmm_adj(h1.reshape(2 * _NP_ALL, HD), a_src, a_dst,
                            a_val))

    image_item, text_item = _stage_a(v_feat, t_feat, item_emb, Wv, bv, Wt,
                                     bt, Wgv, bgv, Wgt, bgt)

    i_src, i_dst, i_val = _pad_edges(image_adj_idx, image_adj_val, _EPAD_KNN,
                                     G)
    t_src, t_dst, t_val = _pad_edges(text_adj_idx, text_adj_val, _EPAD_KNN, G)
    (i_src, i_dst, i_val, t_src, t_dst, t_val, h1) = lax.optimization_barrier(
        (i_src, i_dst, i_val, t_src, t_dst, t_val, h1))
    ii, ti = _spmm_knn(_as_sc_rows(image_item, _NP_ITEM), i_src, i_dst,
                       i_val, _as_sc_rows(text_item, _NP_ITEM), t_src,
                       t_dst, t_val)
    ii = _from_sc(ii)
    ti = _from_sc(ti)

    r_src, r_dst, r_val = _pad_edges(R_idx, R_val, _EPAD_R, G_R)
    iu, tu = _spmm_r(ii.reshape(2 * _NP_ITEM, HD),
                     ti.reshape(2 * _NP_ITEM, HD), r_src, r_dst, r_val)
    iu = _from_sc(iu)
    tu = _from_sc(tu)

    ie = jnp.concatenate([iu[:N_USER], ii[:N_ITEM]], axis=0)
    te = jnp.concatenate([tu[:N_USER], ti[:N_ITEM]], axis=0)
    out = _stage_c(ego, h1[:N_ALL], h2[:N_ALL], ie, te, Wq1, bq1, Wq2, Wip,
                   bip, Wtp, btp)
    return out[:N_USER], out[N_USER:]


# speedup vs baseline: 1.0591x; 1.0127x over previous
import functools

import jax
import jax.numpy as jnp
from jax import lax
from jax.experimental import pallas as pl
from jax.experimental.pallas import tpu as pltpu
from jax.experimental.pallas import tpu_sc as plsc

N_USER = 25000
N_ITEM = 25000
N_ALL = N_USER + N_ITEM
D = 64
HD = 32

L = 16
NC = 2
NS = 16
G = 256
G_R = 128


def _round_up(x, m):
    return (x + m - 1) // m * m


def _zero_spmem(ybuf, rows, s, wc, gsz):
    nfull = wc // gsz
    rem = wc - nfull * gsz

    def zcopy(i, _):
        pltpu.sync_copy(rows, ybuf.at[pl.ds(s * wc + i * gsz, gsz)])
        return 0

    lax.fori_loop(0, nfull, zcopy, 0)
    if rem:
        pltpu.sync_copy(rows.at[pl.ds(0, rem)],
                        ybuf.at[pl.ds(s * wc + nfull * gsz, rem)])


def _zero_rows_vmem(rows, gsz):
    z = jnp.zeros((L,), jnp.float32)

    def zb(i, _):
        for k in range(HD // L):
            rows[i, pl.ds(k * L, L)] = z
        return 0

    lax.fori_loop(0, gsz, zb, 0)


_GATHER_DNUMS = lax.GatherDimensionNumbers(
    offset_dims=(), collapsed_slice_dims=(0,), start_index_map=(0,))


def _lane_broadcast(v16, e):
    return lax.gather(v16, jnp.full((L, 1), e, jnp.int32), _GATHER_DNUMS,
                      slice_sizes=(1,),
                      mode=lax.GatherScatterMode.PROMISE_IN_BOUNDS)


def _transform_src(src3, p, c, sg, gsz):
    for g in range(sg):
        for j in range(gsz // L):
            sl = pl.ds(j * L, L)
            src3[p, g, sl] = src3[p, g, sl] * 2 + c


def _scale_rows(rows, val3, p, g, gsz):
    for j in range(gsz // L):
        v16 = val3[p, g, pl.ds(j * L, L)]
        for e in range(L):
            ee = j * L + e
            vb = _lane_broadcast(v16, e)
            for k in range(HD // L):
                rows[ee, pl.ds(k * L, L)] = rows[ee, pl.ds(k * L, L)] * vb


class _Stream:
    def __init__(self, x_hbm, ybuf, rows_a, rows_b, sem_ga, sem_gb, sem_sa,
                 sem_sb):
        self.x = x_hbm
        self.y = ybuf
        self.rows = (rows_a, rows_b)
        self.sem_g = (sem_ga, sem_gb)
        self.sem_s = (sem_sa, sem_sb)

    def issue_gather(self, src3, p, g, b):
        pltpu.async_copy(self.x.at[src3.at[p, g]], self.rows[b],
                         self.sem_g[b])

    def wait_gather(self, src3, p, g, b):
        pltpu.make_async_copy(self.x.at[src3.at[p, g]], self.rows[b],
                              self.sem_g[b]).wait()

    def issue_scatter(self, dst3, p, g, b):
        pltpu.async_copy(self.rows[b], self.y.at[dst3.at[p, g]],
                         self.sem_s[b], add=True)

    def wait_scatter(self, dst3, p, g, b):
        pltpu.make_async_copy(self.rows[b], self.y.at[dst3.at[p, g]],
                              self.sem_s[b]).wait()


def _edge_phase(streams, src2_h, dst2_h, val2_h, src3, dst3, val3, sem_i,
                s, c, ngrp, sg, gsz):
    nsg = ngrp // sg

    def idx_copies(blk, q):
        row0 = s * ngrp + blk * sg
        return (
            pltpu.make_async_copy(src2_h.at[pl.ds(row0, sg)], src3.at[q],
                                  sem_i),
            pltpu.make_async_copy(dst2_h.at[pl.ds(row0, sg)], dst3.at[q],
                                  sem_i),
            pltpu.make_async_copy(val2_h.at[pl.ds(row0, sg)], val3.at[q],
                                  sem_i),
        )

    for d in idx_copies(0, 0):
        d.start()

    def outer(sb, _):
        p = lax.rem(sb, 2)
        for d in idx_copies(sb, p):
            d.wait()

        @pl.when(sb + 1 < nsg)
        def _():
            for d in idx_copies(sb + 1, 1 - p):
                d.start()

        _transform_src(src3, p, c, sg, gsz)
        for st in streams:
            st.issue_gather(src3, p, 0, 0)

        def inner(t, _):
            g0 = 2 * t
            g1 = 2 * t + 1

            @pl.when(t > 0)
            def _():
                for st in streams:
                    st.wait_scatter(dst3, p, g0 - 1, 1)
            for st in streams:
                st.issue_gather(src3, p, g1, 1)
            for st in streams:
                st.wait_gather(src3, p, g0, 0)
            for st in streams:
                _scale_rows(st.rows[0], val3, p, g0, gsz)
                st.issue_scatter(dst3, p, g0, 0)

            @pl.when(t < sg // 2 - 1)
            def _():
                for st in streams:
                    st.wait_scatter(dst3, p, g0, 0)
                    st.issue_gather(src3, p, g0 + 2, 0)
            for st in streams:
                st.wait_gather(src3, p, g1, 1)
            for st in streams:
                _scale_rows(st.rows[1], val3, p, g1, gsz)
                st.issue_scatter(dst3, p, g1, 1)
            return 0

        lax.fori_loop(0, sg // 2, inner, 0)
        for st in streams:
            st.wait_scatter(dst3, p, sg - 2, 0)
            st.wait_scatter(dst3, p, sg - 1, 1)
        return 0

    lax.fori_loop(0, nsg, outer, 0)


def _writeback(ybuf, out_hbm, s, c, wc):
    pltpu.sync_copy(ybuf.at[pl.ds(s * wc, wc)],
                    out_hbm.at[pl.ds(s * wc, wc), c])


def _npad(n):
    return _round_up(n, 128)


def _make_spmm1(ndst, epad, sg, gsz):
    npad = _npad(ndst)
    wc = npad // NS
    ngrp = epad // NS // gsz
    assert npad % NS == 0 and ngrp % sg == 0 and sg % 2 == 0
    mesh = plsc.VectorSubcoreMesh(core_axis_name="c", subcore_axis_name="s")

    @functools.partial(
        pl.kernel, mesh=mesh,
        compiler_params=pltpu.CompilerParams(use_tc_tiling_on_sc=False),
        out_type=jax.ShapeDtypeStruct((npad, NC, HD), jnp.float32),
        scratch_types=[
            pltpu.VMEM_SHARED((npad, HD), jnp.float32),
            pltpu.VMEM((2, sg, gsz), jnp.int32),
            pltpu.VMEM((2, sg, gsz), jnp.int32),
            pltpu.VMEM((2, sg, gsz), jnp.float32),
            pltpu.VMEM((gsz, HD), jnp.float32),
            pltpu.VMEM((gsz, HD), jnp.float32),
            pltpu.SemaphoreType.DMA,
            pltpu.SemaphoreType.DMA,
            pltpu.SemaphoreType.DMA,
            pltpu.SemaphoreType.DMA,
            pltpu.SemaphoreType.DMA,
        ])
    def k(x_hbm, src_h, dst_h, val_h, out_hbm, ybuf, src3, dst3, val3,
          rows_a, rows_b, sem_i, sem_ga, sem_gb, sem_sa, sem_sb):
        c = lax.axis_index("c")
        s = lax.axis_index("s")
        _zero_rows_vmem(rows_a, gsz)
        _zero_spmem(ybuf, rows_a, s, wc, gsz)
        plsc.subcore_barrier()
        st = _Stream(x_hbm, ybuf, rows_a, rows_b, sem_ga, sem_gb, sem_sa,
                     sem_sb)
        _edge_phase([st], src_h, dst_h, val_h, src3, dst3, val3, sem_i, s,
                    c, ngrp, sg, gsz)
        plsc.subcore_barrier()
        _writeback(ybuf, out_hbm, s, c, wc)

    return k


def _make_spmm2(ndst, epad, sg, gsz, shared_edges):
    npad = _npad(ndst)
    wc = npad // NS
    ngrp = epad // NS // gsz
    assert npad % NS == 0 and ngrp % sg == 0 and sg % 2 == 0
    mesh = plsc.VectorSubcoreMesh(core_axis_name="c", subcore_axis_name="s")

    nrows = 4 if shared_edges else 2
    scratch = [
        pltpu.VMEM_SHARED((npad, HD), jnp.float32),
        pltpu.VMEM_SHARED((npad, HD), jnp.float32),
        pltpu.VMEM((2, sg, gsz), jnp.int32),
        pltpu.VMEM((2, sg, gsz), jnp.int32),
        pltpu.VMEM((2, sg, gsz), jnp.float32),
    ] + [pltpu.VMEM((gsz, HD), jnp.float32)] * nrows \
      + [pltpu.SemaphoreType.DMA] * 9
    out_type = [jax.ShapeDtypeStruct((npad, NC, HD), jnp.float32),
                jax.ShapeDtypeStruct((npad, NC, HD), jnp.float32)]

    if shared_edges:
        @functools.partial(
            pl.kernel, mesh=mesh, out_type=out_type, scratch_types=scratch,
            compiler_params=pltpu.CompilerParams(use_tc_tiling_on_sc=False))
        def k(xa_hbm, xb_hbm, src_h, dst_h, val_h, outa, outb, ybufa, ybufb,
              src3, dst3, val3, rows_aa, rows_ab, rows_ba, rows_bb, sem_i,
              sem_ga1, sem_gb1, sem_sa1, sem_sb1, sem_ga2, sem_gb2, sem_sa2,
              sem_sb2):
            c = lax.axis_index("c")
            s = lax.axis_index("s")
            _zero_rows_vmem(rows_aa, gsz)
            _zero_spmem(ybufa, rows_aa, s, wc, gsz)
            _zero_spmem(ybufb, rows_aa, s, wc, gsz)
            plsc.subcore_barrier()
            sta = _Stream(xa_hbm, ybufa, rows_aa, rows_ab, sem_ga1, sem_gb1,
                          sem_sa1, sem_sb1)
            stb = _Stream(xb_hbm, ybufb, rows_ba, rows_bb, sem_ga2, sem_gb2,
                          sem_sa2, sem_sb2)
            _edge_phase([sta, stb], src_h, dst_h, val_h, src3, dst3, val3,
                        sem_i, s, c, ngrp, sg, gsz)
            plsc.subcore_barrier()
            _writeback(ybufa, outa, s, c, wc)
            _writeback(ybufb, outb, s, c, wc)
    else:
        @functools.partial(
            pl.kernel, mesh=mesh, out_type=out_type, scratch_types=scratch,
            compiler_params=pltpu.CompilerParams(use_tc_tiling_on_sc=False))
        def k(xa_hbm, srca_h, dsta_h, vala_h, xb_hbm, srcb_h, dstb_h,
              valb_h, outa, outb, ybufa, ybufb, src3, dst3, val3, rows_aa,
              rows_ab, sem_i, sem_ga1, sem_gb1, sem_sa1,
              sem_sb1, sem_ga2, sem_gb2, sem_sa2, sem_sb2):
            c = lax.axis_index("c")
            s = lax.axis_index("s")
            _zero_rows_vmem(rows_aa, gsz)
            _zero_spmem(ybufa, rows_aa, s, wc, gsz)
            _zero_spmem(ybufb, rows_aa, s, wc, gsz)
            plsc.subcore_barrier()
            sta = _Stream(xa_hbm, ybufa, rows_aa, rows_ab, sem_ga1, sem_gb1,
                          sem_sa1, sem_sb1)
            stb = _Stream(xb_hbm, ybufb, rows_aa, rows_ab, sem_ga2, sem_gb2,
                          sem_sa2, sem_sb2)
            _edge_phase([sta], srca_h, dsta_h, vala_h, src3, dst3, val3,
                        sem_i, s, c, ngrp, sg, gsz)
            _edge_phase([stb], srcb_h, dstb_h, valb_h, src3, dst3, val3,
                        sem_i, s, c, ngrp, sg, gsz)
            plsc.subcore_barrier()
            _writeback(ybufa, outa, s, c, wc)
            _writeback(ybufb, outb, s, c, wc)

    return k


_BA = 1000
_BC = 2000


def _stage_a_body(vf, tf, ie, Wv, bv, Wt, bt, Wgv, bgv, Wgt, bgt,
                  img_o, txt_o):
    imf = jnp.dot(vf[...], Wv[...], preferred_element_type=jnp.float32)
    imf = imf + bv[...]
    txf = jnp.dot(tf[...], Wt[...], preferred_element_type=jnp.float32)
    txf = txf + bt[...]
    gi = jax.nn.sigmoid(
        jnp.dot(imf, Wgv[...], preferred_element_type=jnp.float32) + bgv[...])
    gt = jax.nn.sigmoid(
        jnp.dot(txf, Wgt[...], preferred_element_type=jnp.float32) + bgt[...])
    img_o[...] = ie[...] * gi
    txt_o[...] = ie[...] * gt


def _stage_a(v_feat, t_feat, item_emb, Wv, bv, Wt, bt, Wgv, bgv, Wgt, bgt):
    grid = (N_ITEM // _BA,)
    row = lambda i: (i, 0)
    full = lambda i: (0, 0)
    return pl.pallas_call(
        _stage_a_body,
        grid=grid,
        in_specs=[
            pl.BlockSpec((_BA, 4096), row),
            pl.BlockSpec((_BA, 384), row),
            pl.BlockSpec((_BA, D), row),
            pl.BlockSpec((4096, D), full),
            pl.BlockSpec((1, D), full),
            pl.BlockSpec((384, D), full),
            pl.BlockSpec((1, D), full),
            pl.BlockSpec((D, D), full),
            pl.BlockSpec((1, D), full),
            pl.BlockSpec((D, D), full),
            pl.BlockSpec((1, D), full),
        ],
        out_specs=[
            pl.BlockSpec((_BA, D), row),
            pl.BlockSpec((_BA, D), row),
        ],
        out_shape=[
            jax.ShapeDtypeStruct((N_ITEM, D), jnp.float32),
            jax.ShapeDtypeStruct((N_ITEM, D), jnp.float32),
        ],
    )(v_feat, t_feat, item_emb, Wv, bv.reshape(1, D), Wt, bt.reshape(1, D),
      Wgv, bgv.reshape(1, D), Wgt, bgt.reshape(1, D))


def _stage_c_body(ego, h1, h2, ie, te, Wq1, bq1, wq2, Wip, bip, Wtp, btp,
                  out):
    content = (ego[...] + h1[...] + h2[...]) * (1.0 / 3.0)
    iev = ie[...]
    tev = te[...]
    q1 = Wq1[...]
    b1 = bq1[...]
    q2 = wq2[...]
    ai = jnp.sum(jnp.tanh(
        jnp.dot(iev, q1, preferred_element_type=jnp.float32) + b1) * q2,
        axis=-1, keepdims=True)
    at = jnp.sum(jnp.tanh(
        jnp.dot(tev, q1, preferred_element_type=jnp.float32) + b1) * q2,
        axis=-1, keepdims=True)
    wi = jax.nn.sigmoid(ai - at)
    common = wi * iev + (1.0 - wi) * tev
    gi = jax.nn.sigmoid(
        jnp.dot(content, Wip[...], preferred_element_type=jnp.float32)
        + bip[...])
    gt = jax.nn.sigmoid(
        jnp.dot(content, Wtp[...], preferred_element_type=jnp.float32)
        + btp[...])
    sep = (iev - common) * gi + (tev - common) * gt
    out[...] = content + (sep + common) * (1.0 / 3.0)


def _stage_c(ego, h1, h2, ie, te, Wq1, bq1, Wq2, Wip, bip, Wtp, btp):
    grid = (N_ALL // _BC,)
    row = lambda i: (i, 0)
    full = lambda i: (0, 0)
    return pl.pallas_call(
        _stage_c_body,
        grid=grid,
        in_specs=[
            pl.BlockSpec((_BC, D), row),
            pl.BlockSpec((_BC, D), row),
            pl.BlockSpec((_BC, D), row),
            pl.BlockSpec((_BC, D), row),
            pl.BlockSpec((_BC, D), row),
            pl.BlockSpec((D, D), full),
            pl.BlockSpec((1, D), full),
            pl.BlockSpec((1, D), full),
            pl.BlockSpec((D, D), full),
            pl.BlockSpec((1, D), full),
            pl.BlockSpec((D, D), full),
            pl.BlockSpec((1, D), full),
        ],
        out_specs=pl.BlockSpec((_BC, D), row),
        out_shape=jax.ShapeDtypeStruct((N_ALL, D), jnp.float32),
    )(ego, h1, h2, ie, te, Wq1, bq1.reshape(1, D), Wq2.reshape(1, D),
      Wip, bip.reshape(1, D), Wtp, btp.reshape(1, D))


_EPAD_ADJ = 819200
_EPAD_KNN = 262144
_EPAD_R = 409600

_NP_ALL = _npad(N_ALL)
_NP_ITEM = _npad(N_ITEM)
_NP_USER = _npad(N_USER)

_spmm_adj = _make_spmm1(N_ALL, _EPAD_ADJ, 8, G)
_spmm_knn = _make_spmm2(N_ITEM, _EPAD_KNN, 8, G, shared_edges=False)
_spmm_r = _make_spmm2(N_USER, _EPAD_R, 8, G_R, shared_edges=True)


def _pad_edges(idx, val, epad, gsz):
    e = val.shape[0]
    pad = epad - e
    src = jnp.pad(idx[1], (0, pad)).reshape(epad // gsz, gsz)
    dst = jnp.pad(idx[0], (0, pad)).reshape(epad // gsz, gsz)
    v = jnp.pad(val, (0, pad)).reshape(epad // gsz, gsz)
    return src, dst, v


def _as_sc_rows(x, npad):
    n = x.shape[0]
    if n < npad:
        x = jnp.pad(x, ((0, npad - n), (0, 0)))
    return x.reshape(2 * npad, HD)


def _from_sc(y):
    return y.reshape(y.shape[0], D)


def kernel(user_emb, item_emb, v_feat, t_feat, adj_idx, adj_val, R_idx,
           R_val, image_adj_idx, image_adj_val, text_adj_idx, text_adj_val,
           Wv, bv, Wt, bt, Wgv, bgv, Wgt, bgt, Wq1, bq1, Wq2, Wip, bip, Wtp,
           btp):
    ego = jnp.concatenate([user_emb, item_emb], axis=0)
    a_src, a_dst, a_val = _pad_edges(adj_idx, adj_val, _EPAD_ADJ, G)
    h1 = _from_sc(_spmm_adj(_as_sc_rows(ego, _NP_ALL), a_src, a_dst, a_val))
    h2 = _from_sc(_spmm_adj(h1.reshape(2 * _NP_ALL, HD), a_src, a_dst,
                            a_val))

    image_item, text_item = _stage_a(v_feat, t_feat, item_emb, Wv, bv, Wt,
                                     bt, Wgv, bgv, Wgt, bgt)

    i_src, i_dst, i_val = _pad_edges(image_adj_idx, image_adj_val, _EPAD_KNN,
                                     G)
    t_src, t_dst, t_val = _pad_edges(text_adj_idx, text_adj_val, _EPAD_KNN, G)
    ii, ti = _spmm_knn(_as_sc_rows(image_item, _NP_ITEM), i_src, i_dst,
                       i_val, _as_sc_rows(text_item, _NP_ITEM), t_src,
                       t_dst, t_val)
    ii = _from_sc(ii)
    ti = _from_sc(ti)

    r_src, r_dst, r_val = _pad_edges(R_idx, R_val, _EPAD_R, G_R)
    iu, tu = _spmm_r(ii.reshape(2 * _NP_ITEM, HD),
                     ti.reshape(2 * _NP_ITEM, HD), r_src, r_dst, r_val)
    iu = _from_sc(iu)
    tu = _from_sc(tu)

    ie = jnp.concatenate([iu[:N_USER], ii[:N_ITEM]], axis=0)
    te = jnp.concatenate([tu[:N_USER], ti[:N_ITEM]], axis=0)
    out = _stage_c(ego, h1[:N_ALL], h2[:N_ALL], ie, te, Wq1, bq1, Wq2, Wip,
                   bip, Wtp, btp)
    return out[:N_USER], out[N_USER:]
